# baseline (speedup 1.0000x reference)
import functools

import jax
import jax.numpy as jnp
from jax import lax
from jax.experimental import pallas as pl
from jax.experimental.pallas import tpu as pltpu
from jax.experimental.pallas import tpu_sc as plsc

F32 = jnp.float32
N, E, B = 10000, 160000, 32
NP, EP = 10240, 163840
ALPHA = 0.5
BN = 1024
BE = 2048
NC, NS = 2, 16
NW = NC * NS
EW = EP // NW
CH = 128
NCH = EW // CH
RPT = NP // NS
SCH = 80
SNCH = EW // SCH


def _sel(x):
    return 1.0507009873554805 * jnp.where(x > 0, x, 1.6732632423543772 * (jnp.exp(x) - 1.0))


def _mm(a, b):
    return jnp.dot(a, b, preferred_element_type=F32)


def _mt(a, b):
    return lax.dot_general(a, b, (((0,), (0,)), ((), ())), preferred_element_type=F32)


def _full(shape):
    return pl.BlockSpec(shape, lambda i: (0,) * len(shape))


def _rows(bn, k):
    return pl.BlockSpec((bn, k), lambda i: (i, 0))


def _bond_body(x, w1, b1, w2, b2, o):
    h = _sel(_mm(x[...], w1[...]) + b1[...])
    o[...] = _sel(_mm(h, w2[...]) + b2[...])


def _node_init_body(a80, s64, afea, cw, oh, aW1, aB1, aW2, aB2, sW1, sB1, sW2,
                    sB2, cW, cB, e1a, e1b, e1d, av_o, sv_o, p_o, q_o, cche_o):
    av = _sel(_mm(_sel(_mm(a80[...], aW1[...]) + aB1[...]), aW2[...]) + aB2[...])
    sv = _sel(_mm(_sel(_mm(s64[...], sW1[...]) + sB1[...]), sW2[...]) + sB2[...])
    av_o[...] = av
    sv_o[...] = sv
    p_o[...] = _mm(av, e1a[...]) + _mm(sv, e1d[...])
    q_o[...] = _mm(av, e1b[...])
    msg = cw[...] * (_mm(afea[...], cW[...]) + cB[...])

    @pl.when(pl.program_id(0) == 0)
    def _():
        cche_o[...] = jnp.zeros_like(cche_o)

    cche_o[...] += _mt(oh[...], msg)


def _edge1_body(x, bw1, bb1, bw2, bb2, gs, gd, w1c, b1, w2, b2, w3, b3,
                h2_o, bvo_o):
    bvx = _sel(_mm(_sel(_mm(x[...], bw1[...]) + bb1[...]), bw2[...]) + bb2[...])
    h1 = _sel(gs[...] + gd[...] + _mm(bvx, w1c[...]) + b1[...])
    h2 = _sel(_mm(h1, w2[...]) + b2[...])
    h2_o[...] = h2
    bvo_o[...] = bvx + ALPHA * (_mm(h2, w3[...]) + b3[...])


def _edge_body(bv, gs, gd, w1c, b1, w2, b2, w3, b3, h2_o, bvo_o):
    bvx = bv[...]
    h1 = _sel(gs[...] + gd[...] + _mm(bvx, w1c[...]) + b1[...])
    h2 = _sel(_mm(h1, w2[...]) + b2[...])
    h2_o[...] = h2
    bvo_o[...] = bvx + ALPHA * (_mm(h2, w3[...]) + b3[...])


def _node_upd_body(av, sv, acc0, acc1, d0, d1,
                   vW1a, vW1b, vW1c, vB1, vW2, vB2, vW3, vB3,
                   uW1a, uW1b, uW1c, uB1, uW2, uB2, uW3, uB3,
                   eW3, eB3, e1a, e1b, e1d, av_o, sv_o, p_o, q_o):
    deg = jnp.maximum(d0[:, 0:1] + d1[:, 0:1], 1.0)
    agg = _mm((acc0[...] + acc1[...]) / deg, eW3[...]) + eB3[...]
    avx = av[...]
    svx = sv[...]
    hv = _sel(_mm(avx, vW1a[...]) + _mm(agg, vW1b[...]) + _mm(svx, vW1c[...]) + vB1[...])
    ache = _mm(_sel(_mm(hv, vW2[...]) + vB2[...]), vW3[...]) + vB3[...]
    hu = _sel(_mm(avx, uW1a[...]) + _mm(agg, uW1b[...]) + _mm(svx, uW1c[...]) + uB1[...])
    sche = _mm(_sel(_mm(hu, uW2[...]) + uB2[...]), uW3[...]) + uB3[...]
    avn = avx + ALPHA * ache
    svn = svx + ALPHA * sche
    av_o[...] = avn
    sv_o[...] = svn
    p_o[...] = _mm(avn, e1a[...]) + _mm(svn, e1d[...])
    q_o[...] = _mm(avn, e1b[...])


def _settf_a_body(x, oh, wp, bp, wl, bl, aV, h_o, s_o, smax_o):
    h = _sel(_mm(x[...], wp[...]) + bp[...])
    for _ in range(3):
        h = _sel(_mm(h, wl[...]) + bl[...])
    s = _mm(h, aV[...])
    h_o[...] = h
    s_o[...] = s
    bm = jnp.max(jnp.where(oh[...] > 0.5, s, -1e30), axis=0, keepdims=True)

    @pl.when(pl.program_id(0) == 0)
    def _():
        smax_o[...] = jnp.full_like(smax_o, -1e30)

    smax_o[...] = jnp.maximum(smax_o[...], bm)


def _settf_h_body(x, wp, bp, wl, bl, aV, h_o, s_o):
    h = _sel(_mm(x[...], wp[...]) + bp[...])
    for _ in range(3):
        h = _sel(_mm(h, wl[...]) + bl[...])
    h_o[...] = h
    s_o[...] = _mm(h, aV[...])


def _segmax_body(s, oh, smax_o):
    bm = jnp.max(jnp.where(oh[...] > 0.5, s[...], -1e30), axis=0, keepdims=True)

    @pl.when(pl.program_id(0) == 0)
    def _():
        smax_o[...] = jnp.full_like(smax_o, -1e30)

    smax_o[...] = jnp.maximum(smax_o[...], bm)


def _settf_b_body(h, s, oh, smax, num_o, den_o, hsum_o, cnt_o):
    ohx = oh[...]
    hx = h[...]
    ssel = jnp.sum(ohx * smax[...], axis=1, keepdims=True)
    ex = jnp.exp(jnp.minimum(s[...] - ssel, 60.0))

    @pl.when(pl.program_id(0) == 0)
    def _():
        num_o[...] = jnp.zeros_like(num_o)
        den_o[...] = jnp.zeros_like(den_o)
        hsum_o[...] = jnp.zeros_like(hsum_o)
        cnt_o[...] = jnp.zeros_like(cnt_o)

    num_o[...] += _mt(ohx, hx * ex)
    den_o[...] += _mt(ohx, ex)
    hsum_o[...] += _mt(ohx, hx)
    cnt_o[...] += _mt(ohx, jnp.ones_like(ex))


def _finalize_body(cche, na, da, ha, ca, nb, db, hb, cb, clW, clB, chWab, chB, t_o):
    logits = _mm(3.0 * cche[...], clW[...]) + clB[...]
    m = jnp.max(logits, axis=0, keepdims=True)
    e = jnp.exp(logits - m)
    comps = e / jnp.sum(e, axis=0, keepdims=True)
    ag = jnp.concatenate([na[...] / jnp.maximum(da[...], 1e-9),
                          ha[...] / jnp.maximum(ca[...], 1.0)], axis=1)
    bg = jnp.concatenate([nb[...] / jnp.maximum(db[...], 1e-9),
                          hb[...] / jnp.maximum(cb[...], 1.0)], axis=1)
    atom_inp = comps * ag
    w = chWab[...]
    t_o[...] = _mm(atom_inp, w[0:256]) + _mm(bg, w[256:512]) + chB[...]


def _final_out_body(oh, sv, t, chWs, o):
    o[...] = _sel(_mm(oh[...], t[...]) + _mm(sv[...], chWs[...]))


_MESH = plsc.VectorSubcoreMesh(core_axis_name="c", subcore_axis_name="s",
                               num_cores=NC, num_subcores=NS)
NH = NCH // 2


def _wid():
    return lax.axis_index("s") * NC + lax.axis_index("c")


@functools.partial(
    pl.kernel, mesh=_MESH,
    out_type=(jax.ShapeDtypeStruct((EP, 128), F32),
              jax.ShapeDtypeStruct((EP, 128), F32)),
    scratch_types=[pltpu.VMEM((EW,), jnp.int32), pltpu.VMEM((EW,), jnp.int32)]
                  + [pltpu.VMEM((CH, 128), F32)] * 6
                  + [pltpu.SemaphoreType.DMA] * 12,
    name="sc_gather2")
def _sc_gather2(tp, tq, src_h, dst_h, out_s, out_d,
                srcv, dstv, pa, qa, pb, qb, pc, qc,
                gpa, gqa, gpb, gqb, gpc, gqc,
                wpa, wqa, wpb, wqb, wpc, wqc):
    base = _wid() * EW
    pltpu.sync_copy(src_h.at[pl.ds(base, EW)], srcv)
    pltpu.sync_copy(dst_h.at[pl.ds(base, EW)], dstv)
    sets = ((pa, qa, gpa, gqa, wpa, wqa),
            (pb, qb, gpb, gqb, wpb, wqb),
            (pc, qc, gpc, gqc, wpc, wqc))

    def fire_g(k, st):
        pltpu.async_copy(tp.at[srcv.at[pl.ds(k * CH, CH)]], st[0], st[2])
        pltpu.async_copy(tq.at[dstv.at[pl.ds(k * CH, CH)]], st[1], st[3])

    def drain_g(st):
        pltpu.make_async_copy(tp.at[pl.ds(0, CH)], st[0], st[2]).wait()
        pltpu.make_async_copy(tq.at[pl.ds(0, CH)], st[1], st[3]).wait()

    def fire_w(k, st):
        pltpu.async_copy(st[0], out_s.at[pl.ds(base + k * CH, CH)], st[4])
        pltpu.async_copy(st[1], out_d.at[pl.ds(base + k * CH, CH)], st[5])

    def drain_w(st):
        pltpu.make_async_copy(st[0], out_s.at[pl.ds(0, CH)], st[4]).wait()
        pltpu.make_async_copy(st[1], out_d.at[pl.ds(0, CH)], st[5]).wait()

    fire_g(0, sets[0])
    fire_g(1, sets[1])

    def body(k, carry):
        def step(j):
            def f():
                st = sets[j]
                drain_g(st)
                fire_w(k, st)
                m = k + 2
                stm = sets[(j + 2) % 3]

                @pl.when(m < NCH)
                def _():
                    @pl.when(m >= 3)
                    def _():
                        drain_w(stm)
                    fire_g(m, stm)
            return f

        lax.switch(k % 3, [step(0), step(1), step(2)])
        return carry

    lax.fori_loop(0, NCH, body, 0)
    for j in range(3):
        drain_w(sets[(NCH - 3 + j) % 3])


@functools.partial(
    pl.kernel, mesh=_MESH,
    out_type=jax.ShapeDtypeStruct((2 * NP, 128), F32),
    scratch_types=[pltpu.VMEM((SNCH, SCH), jnp.int32)]
                  + [pltpu.VMEM((SCH, 128), F32)] * 3
                  + [pltpu.SemaphoreType.DMA] * 6
                  + [pltpu.VMEM_SHARED((NP, 128), F32)],
    name="sc_scatter_add")
def _sc_scatter(h2_h, dst2d_h, zeros_h, out, dstv, ra, rb, rc,
                la, lb, lc, sa, sb, sc_, acc):
    cid = lax.axis_index("c")
    sid = lax.axis_index("s")
    wid = sid * NC + cid
    base = wid * EW
    pltpu.sync_copy(dst2d_h.at[pl.ds(wid * SNCH, SNCH)], dstv)
    pltpu.sync_copy(zeros_h.at[pl.ds(sid * RPT, RPT)], acc.at[pl.ds(sid * RPT, RPT)])
    plsc.subcore_barrier()
    sets = ((ra, la, sa), (rb, lb, sb), (rc, lc, sc_))

    def fire_l(k, st):
        pltpu.async_copy(h2_h.at[pl.ds(base + k * SCH, SCH)], st[0], st[1])

    def drain_l(st):
        pltpu.make_async_copy(h2_h.at[pl.ds(0, SCH)], st[0], st[1]).wait()

    def fire_s(k, st):
        pltpu.async_copy(st[0], acc.at[dstv.at[k]], st[2], add=True)

    def drain_s(st):
        pltpu.make_async_copy(h2_h.at[pl.ds(0, SCH)], st[0], st[2]).wait()

    fire_l(0, sets[0])
    fire_l(1, sets[1])

    def body(k, carry):
        def step(j):
            def f():
                st = sets[j]
                drain_l(st)
                fire_s(k, st)
                m = k + 2
                stm = sets[(j + 2) % 3]

                @pl.when(m < SNCH)
                def _():
                    @pl.when(m >= 3)
                    def _():
                        drain_s(stm)
                    fire_l(m, stm)
            return f

        lax.switch(k % 3, [step(0), step(1), step(2)])
        return carry

    lax.fori_loop(0, SNCH, body, 0)
    for j in range(3):
        drain_s(sets[(SNCH - 3 + j) % 3])
    plsc.subcore_barrier()
    pltpu.sync_copy(acc.at[pl.ds(sid * RPT, RPT)],
                    out.at[pl.ds(cid * NP + sid * RPT, RPT)])


@functools.partial(
    pl.kernel, mesh=_MESH,
    out_type=jax.ShapeDtypeStruct((2 * NP, 128), F32),
    scratch_types=[pltpu.VMEM((NCH, CH), jnp.int32), pltpu.VMEM((CH, 128), F32),
                   pltpu.VMEM_SHARED((NP, 128), F32),
                   pltpu.SemaphoreType.DMA, pltpu.SemaphoreType.DMA],
    name="sc_deg")
def _sc_deg(dst2d_h, zeros_h, ones_h, deg_o, dstv, onesv, dacc, sa, sb):
    cid = lax.axis_index("c")
    sid = lax.axis_index("s")
    wid = sid * NC + cid
    pltpu.sync_copy(dst2d_h.at[pl.ds(wid * NCH, NCH)], dstv)
    pltpu.sync_copy(ones_h, onesv)
    pltpu.sync_copy(zeros_h.at[pl.ds(sid * RPT, RPT)], dacc.at[pl.ds(sid * RPT, RPT)])
    plsc.subcore_barrier()
    sems = (sa, sb)

    def fire(k, s):
        pltpu.async_copy(onesv, dacc.at[dstv.at[k]], s, add=True)

    def drain(s):
        pltpu.make_async_copy(zeros_h.at[pl.ds(0, CH)], onesv, s).wait()

    fire(0, sa)

    def body(k, carry):
        def step(j):
            def f():
                @pl.when(k + 1 < NCH)
                def _():
                    fire(k + 1, sems[1 - j])
                drain(sems[j])
            return f

        lax.switch(k % 2, [step(0), step(1)])
        return carry

    lax.fori_loop(0, NCH, body, 0)
    plsc.subcore_barrier()
    pltpu.sync_copy(dacc.at[pl.ds(sid * RPT, RPT)],
                    deg_o.at[pl.ds(cid * NP + sid * RPT, RPT)])


@functools.partial(
    pl.kernel, mesh=_MESH,
    out_type=jax.ShapeDtypeStruct((EP, 128), F32),
    scratch_types=[pltpu.VMEM((EW,), jnp.int32)]
                  + [pltpu.VMEM((CH, 128), F32)] * 3
                  + [pltpu.SemaphoreType.DMA] * 6,
    name="sc_gather1")
def _sc_gather1(tb, idx_h, out_h, idxv, ba, bb, bc, ga, gb, gc, wa, wb, wc):
    base = _wid() * EW
    pltpu.sync_copy(idx_h.at[pl.ds(base, EW)], idxv)
    sets = ((ba, ga, wa), (bb, gb, wb), (bc, gc, wc))

    def fire_g(k, st):
        pltpu.async_copy(tb.at[idxv.at[pl.ds(k * CH, CH)]], st[0], st[1])

    def drain_g(st):
        pltpu.make_async_copy(tb.at[pl.ds(0, CH)], st[0], st[1]).wait()

    def fire_w(k, st):
        pltpu.async_copy(st[0], out_h.at[pl.ds(base + k * CH, CH)], st[2])

    def drain_w(st):
        pltpu.make_async_copy(st[0], out_h.at[pl.ds(0, CH)], st[2]).wait()

    fire_g(0, sets[0])
    fire_g(1, sets[1])

    def body(k, carry):
        def step(j):
            def f():
                st = sets[j]
                drain_g(st)
                fire_w(k, st)
                m = k + 2
                stm = sets[(j + 2) % 3]

                @pl.when(m < NCH)
                def _():
                    @pl.when(m >= 3)
                    def _():
                        drain_w(stm)
                    fire_g(m, stm)
            return f

        lax.switch(k % 3, [step(0), step(1), step(2)])
        return carry

    lax.fori_loop(0, NCH, body, 0)
    for j in range(3):
        drain_w(sets[(NCH - 3 + j) % 3])


def _bond_mlp(b100, w1, b1, w2, b2):
    g = EP // BE
    return pl.pallas_call(
        _bond_body, grid=(g,),
        in_specs=[_rows(BE, 100), _full((100, 64)), _full((1, 64)),
                  _full((64, 192)), _full((1, 192))],
        out_specs=_rows(BE, 192),
        out_shape=jax.ShapeDtypeStruct((EP, 192), F32),
    )(b100, w1, b1, w2, b2)


def _node_init(a80, s64, afea, cw, oh, ws):
    g = NP // BN
    return pl.pallas_call(
        _node_init_body, grid=(g,),
        in_specs=[_rows(BN, 80), _rows(BN, 64), _rows(BN, 64), _rows(BN, 1),
                  _rows(BN, 32),
                  _full((80, 64)), _full((1, 64)), _full((64, 192)), _full((1, 192)),
                  _full((64, 64)), _full((1, 64)), _full((64, 192)), _full((1, 192)),
                  _full((64, 64)), _full((1, 64)),
                  _full((192, 128)), _full((192, 128)), _full((192, 128))],
        out_specs=[_rows(BN, 192), _rows(BN, 192), _rows(BN, 128), _rows(BN, 128),
                   _full((32, 64))],
        out_shape=[jax.ShapeDtypeStruct((NP, 192), F32),
                   jax.ShapeDtypeStruct((NP, 192), F32),
                   jax.ShapeDtypeStruct((NP, 128), F32),
                   jax.ShapeDtypeStruct((NP, 128), F32),
                   jax.ShapeDtypeStruct((32, 64), F32)],
    )(a80, s64, afea, cw, oh, *ws)


def _edge1_mlp(b100, bw, gs, gd, w1c, b1, w2, b2, w3, b3):
    g = EP // BE
    return pl.pallas_call(
        _edge1_body, grid=(g,),
        in_specs=[_rows(BE, 100), _full((100, 64)), _full((1, 64)),
                  _full((64, 192)), _full((1, 192)),
                  _rows(BE, 128), _rows(BE, 128),
                  _full((192, 128)), _full((1, 128)), _full((128, 128)),
                  _full((1, 128)), _full((128, 192)), _full((1, 192))],
        out_specs=[_rows(BE, 128), _rows(BE, 192)],
        out_shape=[jax.ShapeDtypeStruct((EP, 128), F32),
                   jax.ShapeDtypeStruct((EP, 192), F32)],
    )(b100, *bw, gs, gd, w1c, b1, w2, b2, w3, b3)


def _edge_mlp(bv, gs, gd, w1c, b1, w2, b2, w3, b3):
    g = EP // BE
    return pl.pallas_call(
        _edge_body, grid=(g,),
        in_specs=[_rows(BE, 192), _rows(BE, 128), _rows(BE, 128),
                  _full((192, 128)), _full((1, 128)), _full((128, 128)),
                  _full((1, 128)), _full((128, 192)), _full((1, 192))],
        out_specs=[_rows(BE, 128), _rows(BE, 192)],
        out_shape=[jax.ShapeDtypeStruct((EP, 128), F32),
                   jax.ShapeDtypeStruct((EP, 192), F32)],
    )(bv, gs, gd, w1c, b1, w2, b2, w3, b3)


def _node_update(av, sv, acc0, acc1, d0, d1, ws):
    g = NP // BN
    return pl.pallas_call(
        _node_upd_body, grid=(g,),
        in_specs=[_rows(BN, 192), _rows(BN, 192), _rows(BN, 128), _rows(BN, 128),
                  _rows(BN, 128), _rows(BN, 128),
                  _full((192, 128)), _full((192, 128)), _full((192, 128)),
                  _full((1, 128)), _full((128, 128)), _full((1, 128)),
                  _full((128, 192)), _full((1, 192)),
                  _full((192, 128)), _full((192, 128)), _full((192, 128)),
                  _full((1, 128)), _full((128, 128)), _full((1, 128)),
                  _full((128, 192)), _full((1, 192)),
                  _full((128, 192)), _full((1, 192)),
                  _full((192, 128)), _full((192, 128)), _full((192, 128))],
        out_specs=[_rows(BN, 192), _rows(BN, 192), _rows(BN, 128), _rows(BN, 128)],
        out_shape=[jax.ShapeDtypeStruct((NP, 192), F32),
                   jax.ShapeDtypeStruct((NP, 192), F32),
                   jax.ShapeDtypeStruct((NP, 128), F32),
                   jax.ShapeDtypeStruct((NP, 128), F32)],
    )(av, sv, acc0, acc1, d0, d1, *ws)


def _settf_a(x, oh, wp, bp, wl, bl, aV, rows, bn):
    g = rows // bn
    return pl.pallas_call(
        _settf_a_body, grid=(g,),
        in_specs=[_rows(bn, 192), _rows(bn, 32), _full((192, 128)), _full((1, 128)),
                  _full((128, 128)), _full((1, 128)), _full((128, 1))],
        out_specs=[_rows(bn, 128), _rows(bn, 1), _full((1, 32))],
        out_shape=[jax.ShapeDtypeStruct((rows, 128), F32),
                   jax.ShapeDtypeStruct((rows, 1), F32),
                   jax.ShapeDtypeStruct((1, 32), F32)],
    )(x, oh, wp, bp, wl, bl, aV)


def _settf_h(x, wp, bp, wl, bl, aV, rows, bn):
    g = rows // bn
    return pl.pallas_call(
        _settf_h_body, grid=(g,),
        in_specs=[_rows(bn, 192), _full((192, 128)), _full((1, 128)),
                  _full((128, 128)), _full((1, 128)), _full((128, 1))],
        out_specs=[_rows(bn, 128), _rows(bn, 1)],
        out_shape=[jax.ShapeDtypeStruct((rows, 128), F32),
                   jax.ShapeDtypeStruct((rows, 1), F32)],
    )(x, wp, bp, wl, bl, aV)


def _segmax(s, oh, rows, bn):
    g = rows // bn
    return pl.pallas_call(
        _segmax_body, grid=(g,),
        in_specs=[_rows(bn, 1), _rows(bn, 32)],
        out_specs=_full((1, 32)),
        out_shape=jax.ShapeDtypeStruct((1, 32), F32),
    )(s, oh)


def _settf_b(h, s, oh, smax, rows, bn):
    g = rows // bn
    return pl.pallas_call(
        _settf_b_body, grid=(g,),
        in_specs=[_rows(bn, 128), _rows(bn, 1), _rows(bn, 32), _full((1, 32))],
        out_specs=[_full((32, 128)), _full((32, 1)), _full((32, 128)), _full((32, 1))],
        out_shape=[jax.ShapeDtypeStruct((32, 128), F32),
                   jax.ShapeDtypeStruct((32, 1), F32),
                   jax.ShapeDtypeStruct((32, 128), F32),
                   jax.ShapeDtypeStruct((32, 1), F32)],
    )(h, s, oh, smax)


def _finalize(cche, a4, b4, clW, clB, chWab, chB):
    return pl.pallas_call(
        _finalize_body, grid=(1,),
        in_specs=[_full((32, 64)),
                  _full((32, 128)), _full((32, 1)), _full((32, 128)), _full((32, 1)),
                  _full((32, 128)), _full((32, 1)), _full((32, 128)), _full((32, 1)),
                  _full((64, 1)), _full((1, 1)), _full((512, 256)), _full((1, 256))],
        out_specs=_full((32, 256)),
        out_shape=jax.ShapeDtypeStruct((32, 256), F32),
    )(cche, *a4, *b4, clW, clB, chWab, chB)


def _final_out(oh, sv, t, chWs):
    g = NP // BN
    return pl.pallas_call(
        _final_out_body, grid=(g,),
        in_specs=[_rows(BN, 32), _rows(BN, 192), _full((32, 256)), _full((192, 256))],
        out_specs=_rows(BN, 256),
        out_shape=jax.ShapeDtypeStruct((NP, 256), F32),
    )(oh, sv, t, chWs)


def kernel(atom_vec_embedded, bond_vec_embedded, state_vec_embedded, atom_fea,
           comp_w, params, edge_index, batch):
    p = params

    def padr(x, n):
        return jnp.pad(x, ((0, n - x.shape[0]), (0, 0)))

    a80 = padr(atom_vec_embedded, NP)
    s64 = padr(state_vec_embedded, NP)
    afea = padr(atom_fea, NP)
    cw = padr(comp_w.reshape(-1, 1), NP)
    b100 = padr(bond_vec_embedded, EP)
    batch_p = jnp.pad(batch, (0, NP - N), constant_values=B)
    oh = (batch_p[:, None] == jnp.arange(B, dtype=batch.dtype)).astype(F32)
    src_p = jnp.pad(edge_index[0], (0, EP - E), constant_values=N)
    dst_p = jnp.pad(edge_index[1], (0, EP - E), constant_values=N)
    zeros128 = jnp.zeros((NP, 128), F32)
    ones128 = jnp.ones((CH, 128), F32)

    r2 = lambda b: b.reshape(1, -1)
    e1a, e1b, e1c, e1d = (p['eW1'][0:192], p['eW1'][192:384], p['eW1'][384:576],
                          p['eW1'][576:768])
    vW1a, vW1b, vW1c = p['vW1'][0:192], p['vW1'][192:384], p['vW1'][384:576]
    uW1a, uW1b, uW1c = p['uW1'][0:192], p['uW1'][192:384], p['uW1'][384:576]

    av, sv, P, Q, cche = _node_init(
        a80, s64, afea, cw, oh,
        (p['aW1'], r2(p['aB1']), p['aW2'], r2(p['aB2']),
         p['sW1'], r2(p['sB1']), p['sW2'], r2(p['sB2']),
         p['cW'], r2(p['cB']), e1a, e1b, e1d))

    oh128 = jnp.pad(oh, ((0, 0), (0, 128 - B)))
    dst2d = dst_p.reshape(EP // CH, CH)
    dst2ds = dst_p.reshape(EP // SCH, SCH)
    deg2 = _sc_deg(dst2d, zeros128, ones128)
    d0, d1 = deg2[:NP], deg2[NP:]

    upd_ws = (vW1a, vW1b, vW1c, r2(p['vB1']), p['vW2'], r2(p['vB2']),
              p['vW3'], r2(p['vB3']),
              uW1a, uW1b, uW1c, r2(p['uB1']), p['uW2'], r2(p['uB2']),
              p['uW3'], r2(p['uB3']), p['eW3'], r2(p['eB3']), e1a, e1b, e1d)

    ew = (e1c, r2(p['eB1']), p['eW2'], r2(p['eB2']), p['eW3'], r2(p['eB3']))
    bw = (p['bW1'], r2(p['bB1']), p['bW2'], r2(p['bB2']))
    bv = None
    for it in range(3):
        gs, gd = _sc_gather2(P, Q, src_p, dst_p)
        if it == 0:
            h2, bv = _edge1_mlp(b100, bw, gs, gd, *ew)
        else:
            h2, bv = _edge_mlp(bv, gs, gd, *ew)
        acc2 = _sc_scatter(h2, dst2ds, zeros128)
        av, sv, P, Q = _node_update(av, sv, acc2[:NP], acc2[NP:], d0, d1, upd_ws)

    ohsrc_raw = _sc_gather1(oh128, src_p)
    ohsrc = ohsrc_raw[:, 0:B]
    h_a, s_a, smax_a = _settf_a(av, oh, p['asWp'], r2(p['asBp']), p['asWl'],
                                r2(p['asBl']), p['asA'], NP, BN)
    a4 = _settf_b(h_a, s_a, oh, smax_a, NP, BN)
    h_b, s_b = _settf_h(bv, p['bsWp'], r2(p['bsBp']), p['bsWl'],
                        r2(p['bsBl']), p['bsA'], EP, BE)
    smax_b = _segmax(s_b, ohsrc, EP, BE)
    b4 = _settf_b(h_b, s_b, ohsrc, smax_b, EP, BE)

    t = _finalize(cche, a4, b4, p['clW'], r2(p['clB']), p['chW'][0:512],
                  r2(p['chB']))
    out = _final_out(oh, sv, t, p['chW'][512:704])
    return out[:N]


# speedup vs baseline: 1.0489x; 1.0489x over previous
import functools

import jax
import jax.numpy as jnp
from jax import lax
from jax.experimental import pallas as pl
from jax.experimental.pallas import tpu as pltpu
from jax.experimental.pallas import tpu_sc as plsc

F32 = jnp.float32
N, E, B = 10000, 160000, 32
NP, EP = 10240, 163840
ALPHA = 0.5
BN = 1024
BE = 2048
NC, NS = 2, 16
NW = NC * NS
EW = EP // NW
CH = 128
NCH = EW // CH
RPT = NP // NS
SCH = 80
SNCH = EW // SCH
OCH = 64
ONCH = EW // OCH


def _sel(x):
    return 1.0507009873554805 * jnp.where(x > 0, x, 1.6732632423543772 * (jnp.exp(x) - 1.0))


def _mm(a, b):
    return jnp.dot(a, b, preferred_element_type=F32)


def _mt(a, b):
    return lax.dot_general(a, b, (((0,), (0,)), ((), ())), preferred_element_type=F32)


def _full(shape):
    return pl.BlockSpec(shape, lambda i: (0,) * len(shape))


def _rows(bn, k):
    return pl.BlockSpec((bn, k), lambda i: (i, 0))


def _node_init_body(a80, s64, afea, cw, oh, aW1, aB1, aW2, aB2, sW1, sB1, sW2,
                    sB2, cW, cB, e1a, e1b, e1d, av_o, sv_o, p_o, q_o, cche_o):
    av = _sel(_mm(_sel(_mm(a80[...], aW1[...]) + aB1[...]), aW2[...]) + aB2[...])
    sv = _sel(_mm(_sel(_mm(s64[...], sW1[...]) + sB1[...]), sW2[...]) + sB2[...])
    av_o[...] = av
    sv_o[...] = sv
    p_o[...] = _mm(av, e1a[...]) + _mm(sv, e1d[...])
    q_o[...] = _mm(av, e1b[...])
    msg = cw[...] * (_mm(afea[...], cW[...]) + cB[...])

    @pl.when(pl.program_id(0) == 0)
    def _():
        cche_o[...] = jnp.zeros_like(cche_o)

    cche_o[...] += _mt(oh[...], msg)


def _edge1_body(x, bw1, bb1, bw2, bb2, gs, gd, w1c, b1, w2, b2, w3, b3,
                h2_o, bvo_o):
    bvx = _sel(_mm(_sel(_mm(x[...], bw1[...]) + bb1[...]), bw2[...]) + bb2[...])
    h1 = _sel(gs[...] + gd[...] + _mm(bvx, w1c[...]) + b1[...])
    h2 = _sel(_mm(h1, w2[...]) + b2[...])
    h2_o[...] = h2
    bvo_o[...] = bvx + ALPHA * (_mm(h2, w3[...]) + b3[...])


def _edge_body(bv, gs, gd, w1c, b1, w2, b2, w3, b3, h2_o, bvo_o):
    bvx = bv[...]
    h1 = _sel(gs[...] + gd[...] + _mm(bvx, w1c[...]) + b1[...])
    h2 = _sel(_mm(h1, w2[...]) + b2[...])
    h2_o[...] = h2
    bvo_o[...] = bvx + ALPHA * (_mm(h2, w3[...]) + b3[...])


def _node_upd_body(av, sv, acc0, acc1, d0, d1,
                   vW1a, vW1b, vW1c, vB1, vW2, vB2, vW3, vB3,
                   uW1a, uW1b, uW1c, uB1, uW2, uB2, uW3, uB3,
                   eW3, eB3, e1a, e1b, e1d, av_o, sv_o, p_o, q_o):
    deg = jnp.maximum(d0[:, 0:1] + d1[:, 0:1], 1.0)
    agg = _mm((acc0[...] + acc1[...]) / deg, eW3[...]) + eB3[...]
    avx = av[...]
    svx = sv[...]
    hv = _sel(_mm(avx, vW1a[...]) + _mm(agg, vW1b[...]) + _mm(svx, vW1c[...]) + vB1[...])
    ache = _mm(_sel(_mm(hv, vW2[...]) + vB2[...]), vW3[...]) + vB3[...]
    hu = _sel(_mm(avx, uW1a[...]) + _mm(agg, uW1b[...]) + _mm(svx, uW1c[...]) + uB1[...])
    sche = _mm(_sel(_mm(hu, uW2[...]) + uB2[...]), uW3[...]) + uB3[...]
    avn = avx + ALPHA * ache
    svn = svx + ALPHA * sche
    av_o[...] = avn
    sv_o[...] = svn
    p_o[...] = _mm(avn, e1a[...]) + _mm(svn, e1d[...])
    q_o[...] = _mm(avn, e1b[...])


def _node_upd_last_body(av, sv, acc0, acc1, d0, d1,
                        vW1a, vW1b, vW1c, vB1, vW2, vB2, vW3, vB3,
                        uW1a, uW1b, uW1c, uB1, uW2, uB2, uW3, uB3,
                        eW3, eB3, av_o, sv_o):
    deg = jnp.maximum(d0[:, 0:1] + d1[:, 0:1], 1.0)
    agg = _mm((acc0[...] + acc1[...]) / deg, eW3[...]) + eB3[...]
    avx = av[...]
    svx = sv[...]
    hv = _sel(_mm(avx, vW1a[...]) + _mm(agg, vW1b[...]) + _mm(svx, vW1c[...]) + vB1[...])
    ache = _mm(_sel(_mm(hv, vW2[...]) + vB2[...]), vW3[...]) + vB3[...]
    hu = _sel(_mm(avx, uW1a[...]) + _mm(agg, uW1b[...]) + _mm(svx, uW1c[...]) + uB1[...])
    sche = _mm(_sel(_mm(hu, uW2[...]) + uB2[...]), uW3[...]) + uB3[...]
    av_o[...] = avx + ALPHA * ache
    sv_o[...] = svx + ALPHA * sche


def _settf_a_body(x, oh, wp, bp, wl, bl, aV, h_o, s_o, smax_o):
    h = _sel(_mm(x[...], wp[...]) + bp[...])
    for _ in range(3):
        h = _sel(_mm(h, wl[...]) + bl[...])
    s = _mm(h, aV[...])
    h_o[...] = h
    s_o[...] = s
    bm = jnp.max(jnp.where(oh[...] > 0.5, s, -1e30), axis=0, keepdims=True)

    @pl.when(pl.program_id(0) == 0)
    def _():
        smax_o[...] = jnp.full_like(smax_o, -1e30)

    smax_o[...] = jnp.maximum(smax_o[...], bm)


def _settf_b_body(h, s, oh, smax, num_o, den_o, hsum_o, cnt_o):
    ohx = oh[...]
    hx = h[...]
    ssel = jnp.sum(ohx * smax[...], axis=1, keepdims=True)
    ex = jnp.exp(jnp.minimum(s[...] - ssel, 60.0))

    @pl.when(pl.program_id(0) == 0)
    def _():
        num_o[...] = jnp.zeros_like(num_o)
        den_o[...] = jnp.zeros_like(den_o)
        hsum_o[...] = jnp.zeros_like(hsum_o)
        cnt_o[...] = jnp.zeros_like(cnt_o)

    num_o[...] += _mt(ohx, hx * ex)
    den_o[...] += _mt(ohx, ex)
    hsum_o[...] += _mt(ohx, hx)
    cnt_o[...] += _mt(ohx, jnp.ones_like(ex))


def _finalize_body(cche, na, da, ha, ca, nb, db, hb, cb, clW, clB, chWab, chB, t_o):
    logits = _mm(3.0 * cche[...], clW[...]) + clB[...]
    m = jnp.max(logits, axis=0, keepdims=True)
    e = jnp.exp(logits - m)
    comps = e / jnp.sum(e, axis=0, keepdims=True)
    ag = jnp.concatenate([na[...] / jnp.maximum(da[...], 1e-9),
                          ha[...] / jnp.maximum(ca[...], 1.0)], axis=1)
    bg = jnp.concatenate([nb[...] / jnp.maximum(db[...], 1e-9),
                          hb[...] / jnp.maximum(cb[...], 1.0)], axis=1)
    atom_inp = comps * ag
    w = chWab[...]
    t_o[...] = _mm(atom_inp, w[0:256]) + _mm(bg, w[256:512]) + chB[...]


def _final_out_body(oh, sv, t, chWs, o):
    o[...] = _sel(_mm(oh[...], t[...]) + _mm(sv[...], chWs[...]))


_MESH = plsc.VectorSubcoreMesh(core_axis_name="c", subcore_axis_name="s",
                               num_cores=NC, num_subcores=NS)


def _wid():
    return lax.axis_index("s") * NC + lax.axis_index("c")


@functools.partial(
    pl.kernel, mesh=_MESH,
    out_type=(jax.ShapeDtypeStruct((EP, 128), F32),
              jax.ShapeDtypeStruct((EP, 128), F32)),
    scratch_types=[pltpu.VMEM((EW,), jnp.int32), pltpu.VMEM((EW,), jnp.int32)]
                  + [pltpu.VMEM((CH, 128), F32)] * 6
                  + [pltpu.SemaphoreType.DMA] * 12,
    name="sc_gather2")
def _sc_gather2(tp, tq, src_h, dst_h, out_s, out_d,
                srcv, dstv, pa, qa, pb, qb, pc, qc,
                gpa, gqa, gpb, gqb, gpc, gqc,
                wpa, wqa, wpb, wqb, wpc, wqc):
    base = _wid() * EW
    pltpu.sync_copy(src_h.at[pl.ds(base, EW)], srcv)
    pltpu.sync_copy(dst_h.at[pl.ds(base, EW)], dstv)
    sets = ((pa, qa, gpa, gqa, wpa, wqa),
            (pb, qb, gpb, gqb, wpb, wqb),
            (pc, qc, gpc, gqc, wpc, wqc))

    def fire_g(k, st):
        pltpu.async_copy(tp.at[srcv.at[pl.ds(k * CH, CH)]], st[0], st[2])
        pltpu.async_copy(tq.at[dstv.at[pl.ds(k * CH, CH)]], st[1], st[3])

    def drain_g(st):
        pltpu.make_async_copy(tp.at[pl.ds(0, CH)], st[0], st[2]).wait()
        pltpu.make_async_copy(tq.at[pl.ds(0, CH)], st[1], st[3]).wait()

    def fire_w(k, st):
        pltpu.async_copy(st[0], out_s.at[pl.ds(base + k * CH, CH)], st[4])
        pltpu.async_copy(st[1], out_d.at[pl.ds(base + k * CH, CH)], st[5])

    def drain_w(st):
        pltpu.make_async_copy(st[0], out_s.at[pl.ds(0, CH)], st[4]).wait()
        pltpu.make_async_copy(st[1], out_d.at[pl.ds(0, CH)], st[5]).wait()

    fire_g(0, sets[0])
    fire_g(1, sets[1])

    def body(k, carry):
        def step(j):
            def f():
                st = sets[j]
                drain_g(st)
                fire_w(k, st)
                m = k + 2
                stm = sets[(j + 2) % 3]

                @pl.when(m < NCH)
                def _():
                    @pl.when(m >= 3)
                    def _():
                        drain_w(stm)
                    fire_g(m, stm)
            return f

        lax.switch(k % 3, [step(0), step(1), step(2)])
        return carry

    lax.fori_loop(0, NCH, body, 0)
    for j in range(3):
        drain_w(sets[(NCH - 3 + j) % 3])


@functools.partial(
    pl.kernel, mesh=_MESH,
    out_type=jax.ShapeDtypeStruct((2 * NP, 128), F32),
    scratch_types=[pltpu.VMEM((SNCH, SCH), jnp.int32)]
                  + [pltpu.VMEM((SCH, 128), F32)] * 3
                  + [pltpu.SemaphoreType.DMA] * 6
                  + [pltpu.VMEM_SHARED((NP, 128), F32)],
    name="sc_scatter_add")
def _sc_scatter(h2_h, dst2d_h, zeros_h, out, dstv, ra, rb, rc,
                la, lb, lc, sa, sb, sc_, acc):
    cid = lax.axis_index("c")
    sid = lax.axis_index("s")
    wid = sid * NC + cid
    base = wid * EW
    pltpu.sync_copy(dst2d_h.at[pl.ds(wid * SNCH, SNCH)], dstv)
    pltpu.sync_copy(zeros_h.at[pl.ds(sid * RPT, RPT)], acc.at[pl.ds(sid * RPT, RPT)])
    plsc.subcore_barrier()
    sets = ((ra, la, sa), (rb, lb, sb), (rc, lc, sc_))

    def fire_l(k, st):
        pltpu.async_copy(h2_h.at[pl.ds(base + k * SCH, SCH)], st[0], st[1])

    def drain_l(st):
        pltpu.make_async_copy(h2_h.at[pl.ds(0, SCH)], st[0], st[1]).wait()

    def fire_s(k, st):
        pltpu.async_copy(st[0], acc.at[dstv.at[k]], st[2], add=True)

    def drain_s(st):
        pltpu.make_async_copy(h2_h.at[pl.ds(0, SCH)], st[0], st[2]).wait()

    fire_l(0, sets[0])
    fire_l(1, sets[1])

    def body(k, carry):
        def step(j):
            def f():
                st = sets[j]
                drain_l(st)
                fire_s(k, st)
                m = k + 2
                stm = sets[(j + 2) % 3]

                @pl.when(m < SNCH)
                def _():
                    @pl.when(m >= 3)
                    def _():
                        drain_s(stm)
                    fire_l(m, stm)
            return f

        lax.switch(k % 3, [step(0), step(1), step(2)])
        return carry

    lax.fori_loop(0, SNCH, body, 0)
    for j in range(3):
        drain_s(sets[(SNCH - 3 + j) % 3])
    plsc.subcore_barrier()
    pltpu.sync_copy(acc.at[pl.ds(sid * RPT, RPT)],
                    out.at[pl.ds(cid * NP + sid * RPT, RPT)])


@functools.partial(
    pl.kernel, mesh=_MESH,
    out_type=(jax.ShapeDtypeStruct((2 * NP, 128), F32),
              jax.ShapeDtypeStruct((EP, 128), F32)),
    scratch_types=[pltpu.VMEM((EW,), jnp.int32), pltpu.VMEM((SNCH, SCH), jnp.int32),
                   pltpu.VMEM((SCH, 128), F32),
                   pltpu.VMEM((OCH, 128), F32), pltpu.VMEM((OCH, 128), F32),
                   pltpu.VMEM_SHARED((NP, 128), F32),
                   pltpu.SemaphoreType.DMA, pltpu.SemaphoreType.DMA],
    name="sc_prep")
def _sc_prep(dst2d_h, src_h, oh_h, zeros_h, ones_h, deg_o, ohsrc_o,
             srcv, dstv, onesv, ga, gb, dacc, sga, sgb):
    cid = lax.axis_index("c")
    sid = lax.axis_index("s")
    wid = sid * NC + cid
    base = wid * EW
    pltpu.sync_copy(src_h.at[pl.ds(base, EW)], srcv)
    pltpu.sync_copy(dst2d_h.at[pl.ds(wid * SNCH, SNCH)], dstv)
    pltpu.sync_copy(ones_h, onesv)
    pltpu.sync_copy(zeros_h.at[pl.ds(sid * RPT, RPT)], dacc.at[pl.ds(sid * RPT, RPT)])
    plsc.subcore_barrier()
    gsets = ((ga, sga), (gb, sgb))

    def fire_g(k, st):
        pltpu.async_copy(oh_h.at[srcv.at[pl.ds(k * OCH, OCH)]], st[0], st[1])

    def drain_g(st):
        pltpu.make_async_copy(oh_h.at[pl.ds(0, OCH)], st[0], st[1]).wait()

    fire_g(0, gsets[0])
    fire_g(1, gsets[1])

    def body(k, carry):
        def step(j):
            def f():
                st = gsets[j]
                drain_g(st)
                pltpu.sync_copy(st[0], ohsrc_o.at[pl.ds(base + k * OCH, OCH)])

                @pl.when(k + 2 < ONCH)
                def _():
                    fire_g(k + 2, st)
            return f

        lax.switch(k % 2, [step(0), step(1)])

        @pl.when(k < SNCH)
        def _():
            pltpu.sync_copy(onesv, dacc.at[dstv.at[k]], add=True)
        return carry

    lax.fori_loop(0, ONCH, body, 0)
    plsc.subcore_barrier()
    pltpu.sync_copy(dacc.at[pl.ds(sid * RPT, RPT)],
                    deg_o.at[pl.ds(cid * NP + sid * RPT, RPT)])


def _node_init(a80, s64, afea, cw, oh, ws):
    g = NP // BN
    return pl.pallas_call(
        _node_init_body, grid=(g,),
        in_specs=[_rows(BN, 80), _rows(BN, 64), _rows(BN, 64), _rows(BN, 1),
                  _rows(BN, 32),
                  _full((80, 64)), _full((1, 64)), _full((64, 192)), _full((1, 192)),
                  _full((64, 64)), _full((1, 64)), _full((64, 192)), _full((1, 192)),
                  _full((64, 64)), _full((1, 64)),
                  _full((192, 128)), _full((192, 128)), _full((192, 128))],
        out_specs=[_rows(BN, 192), _rows(BN, 192), _rows(BN, 128), _rows(BN, 128),
                   _full((32, 64))],
        out_shape=[jax.ShapeDtypeStruct((NP, 192), F32),
                   jax.ShapeDtypeStruct((NP, 192), F32),
                   jax.ShapeDtypeStruct((NP, 128), F32),
                   jax.ShapeDtypeStruct((NP, 128), F32),
                   jax.ShapeDtypeStruct((32, 64), F32)],
    )(a80, s64, afea, cw, oh, *ws)


def _edge1_mlp(b100, bw, gs, gd, w1c, b1, w2, b2, w3, b3):
    g = EP // BE
    return pl.pallas_call(
        _edge1_body, grid=(g,),
        in_specs=[_rows(BE, 100), _full((100, 64)), _full((1, 64)),
                  _full((64, 192)), _full((1, 192)),
                  _rows(BE, 128), _rows(BE, 128),
                  _full((192, 128)), _full((1, 128)), _full((128, 128)),
                  _full((1, 128)), _full((128, 192)), _full((1, 192))],
        out_specs=[_rows(BE, 128), _rows(BE, 192)],
        out_shape=[jax.ShapeDtypeStruct((EP, 128), F32),
                   jax.ShapeDtypeStruct((EP, 192), F32)],
    )(b100, *bw, gs, gd, w1c, b1, w2, b2, w3, b3)


def _edge_mlp(bv, gs, gd, w1c, b1, w2, b2, w3, b3):
    g = EP // BE
    return pl.pallas_call(
        _edge_body, grid=(g,),
        in_specs=[_rows(BE, 192), _rows(BE, 128), _rows(BE, 128),
                  _full((192, 128)), _full((1, 128)), _full((128, 128)),
                  _full((1, 128)), _full((128, 192)), _full((1, 192))],
        out_specs=[_rows(BE, 128), _rows(BE, 192)],
        out_shape=[jax.ShapeDtypeStruct((EP, 128), F32),
                   jax.ShapeDtypeStruct((EP, 192), F32)],
    )(bv, gs, gd, w1c, b1, w2, b2, w3, b3)


def _node_update(av, sv, acc0, acc1, d0, d1, ws):
    g = NP // BN
    return pl.pallas_call(
        _node_upd_body, grid=(g,),
        in_specs=[_rows(BN, 192), _rows(BN, 192), _rows(BN, 128), _rows(BN, 128),
                  _rows(BN, 128), _rows(BN, 128),
                  _full((192, 128)), _full((192, 128)), _full((192, 128)),
                  _full((1, 128)), _full((128, 128)), _full((1, 128)),
                  _full((128, 192)), _full((1, 192)),
                  _full((192, 128)), _full((192, 128)), _full((192, 128)),
                  _full((1, 128)), _full((128, 128)), _full((1, 128)),
                  _full((128, 192)), _full((1, 192)),
                  _full((128, 192)), _full((1, 192)),
                  _full((192, 128)), _full((192, 128)), _full((192, 128))],
        out_specs=[_rows(BN, 192), _rows(BN, 192), _rows(BN, 128), _rows(BN, 128)],
        out_shape=[jax.ShapeDtypeStruct((NP, 192), F32),
                   jax.ShapeDtypeStruct((NP, 192), F32),
                   jax.ShapeDtypeStruct((NP, 128), F32),
                   jax.ShapeDtypeStruct((NP, 128), F32)],
    )(av, sv, acc0, acc1, d0, d1, *ws)


def _node_update_last(av, sv, acc0, acc1, d0, d1, ws):
    g = NP // BN
    return pl.pallas_call(
        _node_upd_last_body, grid=(g,),
        in_specs=[_rows(BN, 192), _rows(BN, 192), _rows(BN, 128), _rows(BN, 128),
                  _rows(BN, 128), _rows(BN, 128),
                  _full((192, 128)), _full((192, 128)), _full((192, 128)),
                  _full((1, 128)), _full((128, 128)), _full((1, 128)),
                  _full((128, 192)), _full((1, 192)),
                  _full((192, 128)), _full((192, 128)), _full((192, 128)),
                  _full((1, 128)), _full((128, 128)), _full((1, 128)),
                  _full((128, 192)), _full((1, 192)),
                  _full((128, 192)), _full((1, 192))],
        out_specs=[_rows(BN, 192), _rows(BN, 192)],
        out_shape=[jax.ShapeDtypeStruct((NP, 192), F32),
                   jax.ShapeDtypeStruct((NP, 192), F32)],
    )(av, sv, acc0, acc1, d0, d1, *ws[:-3])


def _settf_a(x, oh, wp, bp, wl, bl, aV, rows, bn):
    g = rows // bn
    return pl.pallas_call(
        _settf_a_body, grid=(g,),
        in_specs=[_rows(bn, 192), _rows(bn, 32), _full((192, 128)), _full((1, 128)),
                  _full((128, 128)), _full((1, 128)), _full((128, 1))],
        out_specs=[_rows(bn, 128), _rows(bn, 1), _full((1, 32))],
        out_shape=[jax.ShapeDtypeStruct((rows, 128), F32),
                   jax.ShapeDtypeStruct((rows, 1), F32),
                   jax.ShapeDtypeStruct((1, 32), F32)],
    )(x, oh, wp, bp, wl, bl, aV)


def _settf_b(h, s, oh, smax, rows, bn):
    g = rows // bn
    return pl.pallas_call(
        _settf_b_body, grid=(g,),
        in_specs=[_rows(bn, 128), _rows(bn, 1), _rows(bn, 32), _full((1, 32))],
        out_specs=[_full((32, 128)), _full((32, 1)), _full((32, 128)), _full((32, 1))],
        out_shape=[jax.ShapeDtypeStruct((32, 128), F32),
                   jax.ShapeDtypeStruct((32, 1), F32),
                   jax.ShapeDtypeStruct((32, 128), F32),
                   jax.ShapeDtypeStruct((32, 1), F32)],
    )(h, s, oh, smax)


def _finalize(cche, a4, b4, clW, clB, chWab, chB):
    return pl.pallas_call(
        _finalize_body, grid=(1,),
        in_specs=[_full((32, 64)),
                  _full((32, 128)), _full((32, 1)), _full((32, 128)), _full((32, 1)),
                  _full((32, 128)), _full((32, 1)), _full((32, 128)), _full((32, 1)),
                  _full((64, 1)), _full((1, 1)), _full((512, 256)), _full((1, 256))],
        out_specs=_full((32, 256)),
        out_shape=jax.ShapeDtypeStruct((32, 256), F32),
    )(cche, *a4, *b4, clW, clB, chWab, chB)


def _final_out(oh, sv, t, chWs):
    g = NP // BN
    return pl.pallas_call(
        _final_out_body, grid=(g,),
        in_specs=[_rows(BN, 32), _rows(BN, 192), _full((32, 256)), _full((192, 256))],
        out_specs=_rows(BN, 256),
        out_shape=jax.ShapeDtypeStruct((NP, 256), F32),
    )(oh, sv, t, chWs)


def kernel(atom_vec_embedded, bond_vec_embedded, state_vec_embedded, atom_fea,
           comp_w, params, edge_index, batch):
    p = params

    def padr(x, n):
        return jnp.pad(x, ((0, n - x.shape[0]), (0, 0)))

    a80 = padr(atom_vec_embedded, NP)
    s64 = padr(state_vec_embedded, NP)
    afea = padr(atom_fea, NP)
    cw = padr(comp_w.reshape(-1, 1), NP)
    b100 = padr(bond_vec_embedded, EP)
    batch_p = jnp.pad(batch, (0, NP - N), constant_values=B)
    oh = (batch_p[:, None] == jnp.arange(B, dtype=batch.dtype)).astype(F32)
    src_p = jnp.pad(edge_index[0], (0, EP - E), constant_values=N)
    dst_p = jnp.pad(edge_index[1], (0, EP - E), constant_values=N)
    zeros128 = jnp.zeros((NP, 128), F32)
    ones128 = jnp.ones((SCH, 128), F32)

    r2 = lambda b: b.reshape(1, -1)
    e1a, e1b, e1c, e1d = (p['eW1'][0:192], p['eW1'][192:384], p['eW1'][384:576],
                          p['eW1'][576:768])
    vW1a, vW1b, vW1c = p['vW1'][0:192], p['vW1'][192:384], p['vW1'][384:576]
    uW1a, uW1b, uW1c = p['uW1'][0:192], p['uW1'][192:384], p['uW1'][384:576]

    av, sv, P, Q, cche = _node_init(
        a80, s64, afea, cw, oh,
        (p['aW1'], r2(p['aB1']), p['aW2'], r2(p['aB2']),
         p['sW1'], r2(p['sB1']), p['sW2'], r2(p['sB2']),
         p['cW'], r2(p['cB']), e1a, e1b, e1d))

    oh128 = jnp.pad(oh, ((0, 0), (0, 128 - B)))
    dst2ds = dst_p.reshape(EP // SCH, SCH)
    deg2, ohsrc_raw = _sc_prep(dst2ds, src_p, oh128, zeros128, ones128)
    ohsrc = ohsrc_raw[:, 0:B]
    d0, d1 = deg2[:NP], deg2[NP:]

    upd_ws = (vW1a, vW1b, vW1c, r2(p['vB1']), p['vW2'], r2(p['vB2']),
              p['vW3'], r2(p['vB3']),
              uW1a, uW1b, uW1c, r2(p['uB1']), p['uW2'], r2(p['uB2']),
              p['uW3'], r2(p['uB3']), p['eW3'], r2(p['eB3']), e1a, e1b, e1d)

    ew = (e1c, r2(p['eB1']), p['eW2'], r2(p['eB2']), p['eW3'], r2(p['eB3']))
    bw = (p['bW1'], r2(p['bB1']), p['bW2'], r2(p['bB2']))
    bv = None
    for it in range(3):
        gs, gd = _sc_gather2(P, Q, src_p, dst_p)
        if it == 0:
            h2, bv = _edge1_mlp(b100, bw, gs, gd, *ew)
        else:
            h2, bv = _edge_mlp(bv, gs, gd, *ew)
        acc2 = _sc_scatter(h2, dst2ds, zeros128)
        if it < 2:
            av, sv, P, Q = _node_update(av, sv, acc2[:NP], acc2[NP:], d0, d1, upd_ws)
        else:
            av, sv = _node_update_last(av, sv, acc2[:NP], acc2[NP:], d0, d1, upd_ws)

    h_a, s_a, smax_a = _settf_a(av, oh, p['asWp'], r2(p['asBp']), p['asWl'],
                                r2(p['asBl']), p['asA'], NP, BN)
    a4 = _settf_b(h_a, s_a, oh, smax_a, NP, BN)
    h_b, s_b, smax_b = _settf_a(bv, ohsrc, p['bsWp'], r2(p['bsBp']), p['bsWl'],
                                r2(p['bsBl']), p['bsA'], EP, BE)
    b4 = _settf_b(h_b, s_b, ohsrc, smax_b, EP, BE)

    t = _finalize(cche, a4, b4, p['clW'], r2(p['clB']), p['chW'][0:512],
                  r2(p['chB']))
    out = _final_out(oh, sv, t, p['chW'][512:704])
    return out[:N]


# speedup vs baseline: 1.0841x; 1.0336x over previous
import functools

import jax
import jax.numpy as jnp
from jax import lax
from jax.experimental import pallas as pl
from jax.experimental.pallas import tpu as pltpu
from jax.experimental.pallas import tpu_sc as plsc

F32 = jnp.float32
N, E, B = 10000, 160000, 32
NP, EP = 10240, 163840
ALPHA = 0.5
BN = 1024
BE = 4096
NC, NS = 2, 16
NW = NC * NS
EW = EP // NW
CH = 128
NCH = EW // CH
RPT = NP // NS
SCH = 80
SNCH = EW // SCH
OCH = 64
ONCH = EW // OCH


def _sel(x):
    return 1.0507009873554805 * jnp.where(x > 0, x, 1.6732632423543772 * (jnp.exp(x) - 1.0))


def _mm(a, b):
    return jnp.dot(a, b, preferred_element_type=F32)


def _mt(a, b):
    return lax.dot_general(a, b, (((0,), (0,)), ((), ())), preferred_element_type=F32)


def _full(shape):
    return pl.BlockSpec(shape, lambda i: (0,) * len(shape))


def _rows(bn, k):
    return pl.BlockSpec((bn, k), lambda i: (i, 0))


def _node_init_body(a80, s64, afea, cw, oh, aW1, aB1, aW2, aB2, sW1, sB1, sW2,
                    sB2, cW, cB, e1a, e1b, e1d, av_o, sv_o, p_o, q_o, cche_o):
    av = _sel(_mm(_sel(_mm(a80[...], aW1[...]) + aB1[...]), aW2[...]) + aB2[...])
    sv = _sel(_mm(_sel(_mm(s64[...], sW1[...]) + sB1[...]), sW2[...]) + sB2[...])
    av_o[...] = av
    sv_o[...] = sv
    p_o[...] = _mm(av, e1a[...]) + _mm(sv, e1d[...])
    q_o[...] = _mm(av, e1b[...])
    msg = cw[...] * (_mm(afea[...], cW[...]) + cB[...])

    @pl.when(pl.program_id(0) == 0)
    def _():
        cche_o[...] = jnp.zeros_like(cche_o)

    cche_o[...] += _mt(oh[...], msg)


def _edge1_body(x, bw1, bb1, bw2, bb2, gs, gd, w1c, b1, w2, b2, w3, b3,
                h2_o, bvo_o):
    bvx = _sel(_mm(_sel(_mm(x[...], bw1[...]) + bb1[...]), bw2[...]) + bb2[...])
    h1 = _sel(gs[...] + gd[...] + _mm(bvx, w1c[...]) + b1[...])
    h2 = _sel(_mm(h1, w2[...]) + b2[...])
    h2_o[...] = h2
    bvo_o[...] = bvx + ALPHA * (_mm(h2, w3[...]) + b3[...])


def _edge_body(bv, gs, gd, w1c, b1, w2, b2, w3, b3, h2_o, bvo_o):
    bvx = bv[...]
    h1 = _sel(gs[...] + gd[...] + _mm(bvx, w1c[...]) + b1[...])
    h2 = _sel(_mm(h1, w2[...]) + b2[...])
    h2_o[...] = h2
    bvo_o[...] = bvx + ALPHA * (_mm(h2, w3[...]) + b3[...])


def _node_upd_body(av, sv, acc0, acc1, d0, d1,
                   vW1a, vW1b, vW1c, vB1, vW2, vB2, vW3, vB3,
                   uW1a, uW1b, uW1c, uB1, uW2, uB2, uW3, uB3,
                   eW3, eB3, e1a, e1b, e1d, av_o, sv_o, p_o, q_o):
    deg = jnp.maximum(d0[:, 0:1] + d1[:, 0:1], 1.0)
    agg = _mm((acc0[...] + acc1[...]) / deg, eW3[...]) + eB3[...]
    avx = av[...]
    svx = sv[...]
    hv = _sel(_mm(avx, vW1a[...]) + _mm(agg, vW1b[...]) + _mm(svx, vW1c[...]) + vB1[...])
    ache = _mm(_sel(_mm(hv, vW2[...]) + vB2[...]), vW3[...]) + vB3[...]
    hu = _sel(_mm(avx, uW1a[...]) + _mm(agg, uW1b[...]) + _mm(svx, uW1c[...]) + uB1[...])
    sche = _mm(_sel(_mm(hu, uW2[...]) + uB2[...]), uW3[...]) + uB3[...]
    avn = avx + ALPHA * ache
    svn = svx + ALPHA * sche
    av_o[...] = avn
    sv_o[...] = svn
    p_o[...] = _mm(avn, e1a[...]) + _mm(svn, e1d[...])
    q_o[...] = _mm(avn, e1b[...])


def _node_upd_last_body(av, sv, acc0, acc1, d0, d1,
                        vW1a, vW1b, vW1c, vB1, vW2, vB2, vW3, vB3,
                        uW1a, uW1b, uW1c, uB1, uW2, uB2, uW3, uB3,
                        eW3, eB3, av_o, sv_o):
    deg = jnp.maximum(d0[:, 0:1] + d1[:, 0:1], 1.0)
    agg = _mm((acc0[...] + acc1[...]) / deg, eW3[...]) + eB3[...]
    avx = av[...]
    svx = sv[...]
    hv = _sel(_mm(avx, vW1a[...]) + _mm(agg, vW1b[...]) + _mm(svx, vW1c[...]) + vB1[...])
    ache = _mm(_sel(_mm(hv, vW2[...]) + vB2[...]), vW3[...]) + vB3[...]
    hu = _sel(_mm(avx, uW1a[...]) + _mm(agg, uW1b[...]) + _mm(svx, uW1c[...]) + uB1[...])
    sche = _mm(_sel(_mm(hu, uW2[...]) + uB2[...]), uW3[...]) + uB3[...]
    av_o[...] = avx + ALPHA * ache
    sv_o[...] = svx + ALPHA * sche


def _settf_a_body(x, oh, wp, bp, wl, bl, aV, h_o, s_o, smax_o):
    h = _sel(_mm(x[...], wp[...]) + bp[...])
    for _ in range(3):
        h = _sel(_mm(h, wl[...]) + bl[...])
    s = _mm(h, aV[...])
    h_o[...] = h
    s_o[...] = s
    bm = jnp.max(jnp.where(oh[...] > 0.5, s, -1e30), axis=0, keepdims=True)

    @pl.when(pl.program_id(0) == 0)
    def _():
        smax_o[...] = jnp.full_like(smax_o, -1e30)

    smax_o[...] = jnp.maximum(smax_o[...], bm)


def _settf_b_body(h, s, oh, smax, num_o, den_o, hsum_o, cnt_o):
    ohx = oh[...]
    hx = h[...]
    ssel = jnp.sum(ohx * smax[...], axis=1, keepdims=True)
    ex = jnp.exp(jnp.minimum(s[...] - ssel, 60.0))

    @pl.when(pl.program_id(0) == 0)
    def _():
        num_o[...] = jnp.zeros_like(num_o)
        den_o[...] = jnp.zeros_like(den_o)
        hsum_o[...] = jnp.zeros_like(hsum_o)
        cnt_o[...] = jnp.zeros_like(cnt_o)

    num_o[...] += _mt(ohx, hx * ex)
    den_o[...] += _mt(ohx, ex)
    hsum_o[...] += _mt(ohx, hx)
    cnt_o[...] += _mt(ohx, jnp.ones_like(ex))


def _finalize_body(cche, na, da, ha, ca, nb, db, hb, cb, clW, clB, chWab, chB, t_o):
    logits = _mm(3.0 * cche[...], clW[...]) + clB[...]
    m = jnp.max(logits, axis=0, keepdims=True)
    e = jnp.exp(logits - m)
    comps = e / jnp.sum(e, axis=0, keepdims=True)
    ag = jnp.concatenate([na[...] / jnp.maximum(da[...], 1e-9),
                          ha[...] / jnp.maximum(ca[...], 1.0)], axis=1)
    bg = jnp.concatenate([nb[...] / jnp.maximum(db[...], 1e-9),
                          hb[...] / jnp.maximum(cb[...], 1.0)], axis=1)
    atom_inp = comps * ag
    w = chWab[...]
    t_o[...] = _mm(atom_inp, w[0:256]) + _mm(bg, w[256:512]) + chB[...]


def _final_out_body(oh, sv, t, chWs, o):
    o[...] = _sel(_mm(oh[...], t[...]) + _mm(sv[...], chWs[...]))


_MESH = plsc.VectorSubcoreMesh(core_axis_name="c", subcore_axis_name="s",
                               num_cores=NC, num_subcores=NS)


def _wid():
    return lax.axis_index("s") * NC + lax.axis_index("c")


@functools.partial(
    pl.kernel, mesh=_MESH,
    out_type=(jax.ShapeDtypeStruct((EP, 128), F32),
              jax.ShapeDtypeStruct((EP, 128), F32)),
    scratch_types=[pltpu.VMEM((EW,), jnp.int32), pltpu.VMEM((EW,), jnp.int32)]
                  + [pltpu.VMEM((CH, 128), F32)] * 6
                  + [pltpu.SemaphoreType.DMA] * 12,
    name="sc_gather2")
def _sc_gather2(tp, tq, src_h, dst_h, out_s, out_d,
                srcv, dstv, pa, qa, pb, qb, pc, qc,
                gpa, gqa, gpb, gqb, gpc, gqc,
                wpa, wqa, wpb, wqb, wpc, wqc):
    base = _wid() * EW
    pltpu.sync_copy(src_h.at[pl.ds(base, EW)], srcv)
    pltpu.sync_copy(dst_h.at[pl.ds(base, EW)], dstv)
    sets = ((pa, qa, gpa, gqa, wpa, wqa),
            (pb, qb, gpb, gqb, wpb, wqb),
            (pc, qc, gpc, gqc, wpc, wqc))

    def fire_g(k, st):
        pltpu.async_copy(tp.at[srcv.at[pl.ds(k * CH, CH)]], st[0], st[2])
        pltpu.async_copy(tq.at[dstv.at[pl.ds(k * CH, CH)]], st[1], st[3])

    def drain_g(st):
        pltpu.make_async_copy(tp.at[pl.ds(0, CH)], st[0], st[2]).wait()
        pltpu.make_async_copy(tq.at[pl.ds(0, CH)], st[1], st[3]).wait()

    def fire_w(k, st):
        pltpu.async_copy(st[0], out_s.at[pl.ds(base + k * CH, CH)], st[4])
        pltpu.async_copy(st[1], out_d.at[pl.ds(base + k * CH, CH)], st[5])

    def drain_w(st):
        pltpu.make_async_copy(st[0], out_s.at[pl.ds(0, CH)], st[4]).wait()
        pltpu.make_async_copy(st[1], out_d.at[pl.ds(0, CH)], st[5]).wait()

    fire_g(0, sets[0])
    fire_g(1, sets[1])

    def body(k, carry):
        def step(j):
            def f():
                st = sets[j]
                drain_g(st)
                fire_w(k, st)
                m = k + 2
                stm = sets[(j + 2) % 3]

                @pl.when(m < NCH)
                def _():
                    @pl.when(m >= 3)
                    def _():
                        drain_w(stm)
                    fire_g(m, stm)
            return f

        lax.switch(k % 3, [step(0), step(1), step(2)])
        return carry

    lax.fori_loop(0, NCH, body, 0)
    for j in range(3):
        drain_w(sets[(NCH - 3 + j) % 3])


@functools.partial(
    pl.kernel, mesh=_MESH,
    out_type=jax.ShapeDtypeStruct((2 * NP, 128), F32),
    scratch_types=[pltpu.VMEM((SNCH, SCH), jnp.int32)]
                  + [pltpu.VMEM((SCH, 128), F32)] * 3
                  + [pltpu.SemaphoreType.DMA] * 6
                  + [pltpu.VMEM_SHARED((NP, 128), F32)],
    name="sc_scatter_add")
def _sc_scatter(h2_h, dst2d_h, zeros_h, out, dstv, ra, rb, rc,
                la, lb, lc, sa, sb, sc_, acc):
    cid = lax.axis_index("c")
    sid = lax.axis_index("s")
    wid = sid * NC + cid
    base = wid * EW
    pltpu.sync_copy(dst2d_h.at[pl.ds(wid * SNCH, SNCH)], dstv)
    pltpu.sync_copy(zeros_h.at[pl.ds(sid * RPT, RPT)], acc.at[pl.ds(sid * RPT, RPT)])
    plsc.subcore_barrier()
    sets = ((ra, la, sa), (rb, lb, sb), (rc, lc, sc_))

    def fire_l(k, st):
        pltpu.async_copy(h2_h.at[pl.ds(base + k * SCH, SCH)], st[0], st[1])

    def drain_l(st):
        pltpu.make_async_copy(h2_h.at[pl.ds(0, SCH)], st[0], st[1]).wait()

    def fire_s(k, st):
        pltpu.async_copy(st[0], acc.at[dstv.at[k]], st[2], add=True)

    def drain_s(st):
        pltpu.make_async_copy(h2_h.at[pl.ds(0, SCH)], st[0], st[2]).wait()

    fire_l(0, sets[0])
    fire_l(1, sets[1])

    def body(k, carry):
        def step(j):
            def f():
                st = sets[j]
                drain_l(st)
                fire_s(k, st)
                m = k + 2
                stm = sets[(j + 2) % 3]

                @pl.when(m < SNCH)
                def _():
                    @pl.when(m >= 3)
                    def _():
                        drain_s(stm)
                    fire_l(m, stm)
            return f

        lax.switch(k % 3, [step(0), step(1), step(2)])
        return carry

    lax.fori_loop(0, SNCH, body, 0)
    for j in range(3):
        drain_s(sets[(SNCH - 3 + j) % 3])
    plsc.subcore_barrier()
    pltpu.sync_copy(acc.at[pl.ds(sid * RPT, RPT)],
                    out.at[pl.ds(cid * NP + sid * RPT, RPT)])


@functools.partial(
    pl.kernel, mesh=_MESH,
    out_type=(jax.ShapeDtypeStruct((2 * NP, 128), F32),
              jax.ShapeDtypeStruct((EP, 128), F32)),
    scratch_types=[pltpu.VMEM((EW,), jnp.int32), pltpu.VMEM((SNCH, SCH), jnp.int32),
                   pltpu.VMEM((SCH, 128), F32),
                   pltpu.VMEM((OCH, 128), F32), pltpu.VMEM((OCH, 128), F32),
                   pltpu.VMEM_SHARED((NP, 128), F32),
                   pltpu.SemaphoreType.DMA, pltpu.SemaphoreType.DMA],
    name="sc_prep")
def _sc_prep(dst2d_h, src_h, oh_h, zeros_h, ones_h, deg_o, ohsrc_o,
             srcv, dstv, onesv, ga, gb, dacc, sga, sgb):
    cid = lax.axis_index("c")
    sid = lax.axis_index("s")
    wid = sid * NC + cid
    base = wid * EW
    pltpu.sync_copy(src_h.at[pl.ds(base, EW)], srcv)
    pltpu.sync_copy(dst2d_h.at[pl.ds(wid * SNCH, SNCH)], dstv)
    pltpu.sync_copy(ones_h, onesv)
    pltpu.sync_copy(zeros_h.at[pl.ds(sid * RPT, RPT)], dacc.at[pl.ds(sid * RPT, RPT)])
    plsc.subcore_barrier()
    gsets = ((ga, sga), (gb, sgb))

    def fire_g(k, st):
        pltpu.async_copy(oh_h.at[srcv.at[pl.ds(k * OCH, OCH)]], st[0], st[1])

    def drain_g(st):
        pltpu.make_async_copy(oh_h.at[pl.ds(0, OCH)], st[0], st[1]).wait()

    fire_g(0, gsets[0])
    fire_g(1, gsets[1])

    def body(k, carry):
        def step(j):
            def f():
                st = gsets[j]
                drain_g(st)
                pltpu.sync_copy(st[0], ohsrc_o.at[pl.ds(base + k * OCH, OCH)])

                @pl.when(k + 2 < ONCH)
                def _():
                    fire_g(k + 2, st)
            return f

        lax.switch(k % 2, [step(0), step(1)])

        @pl.when(k < SNCH)
        def _():
            pltpu.sync_copy(onesv, dacc.at[dstv.at[k]], add=True)
        return carry

    lax.fori_loop(0, ONCH, body, 0)
    plsc.subcore_barrier()
    pltpu.sync_copy(dacc.at[pl.ds(sid * RPT, RPT)],
                    deg_o.at[pl.ds(cid * NP + sid * RPT, RPT)])


def _node_init(a80, s64, afea, cw, oh, ws):
    g = NP // BN
    return pl.pallas_call(
        _node_init_body, grid=(g,),
        in_specs=[_rows(BN, 80), _rows(BN, 64), _rows(BN, 64), _rows(BN, 1),
                  _rows(BN, 32),
                  _full((80, 64)), _full((1, 64)), _full((64, 192)), _full((1, 192)),
                  _full((64, 64)), _full((1, 64)), _full((64, 192)), _full((1, 192)),
                  _full((64, 64)), _full((1, 64)),
                  _full((192, 128)), _full((192, 128)), _full((192, 128))],
        out_specs=[_rows(BN, 192), _rows(BN, 192), _rows(BN, 128), _rows(BN, 128),
                   _full((32, 64))],
        out_shape=[jax.ShapeDtypeStruct((NP, 192), F32),
                   jax.ShapeDtypeStruct((NP, 192), F32),
                   jax.ShapeDtypeStruct((NP, 128), F32),
                   jax.ShapeDtypeStruct((NP, 128), F32),
                   jax.ShapeDtypeStruct((32, 64), F32)],
    )(a80, s64, afea, cw, oh, *ws)


def _edge1_mlp(b100, bw, gs, gd, w1c, b1, w2, b2, w3, b3):
    g = EP // BE
    return pl.pallas_call(
        _edge1_body, grid=(g,),
        in_specs=[_rows(BE, 100), _full((100, 64)), _full((1, 64)),
                  _full((64, 192)), _full((1, 192)),
                  _rows(BE, 128), _rows(BE, 128),
                  _full((192, 128)), _full((1, 128)), _full((128, 128)),
                  _full((1, 128)), _full((128, 192)), _full((1, 192))],
        out_specs=[_rows(BE, 128), _rows(BE, 192)],
        out_shape=[jax.ShapeDtypeStruct((EP, 128), F32),
                   jax.ShapeDtypeStruct((EP, 192), F32)],
    )(b100, *bw, gs, gd, w1c, b1, w2, b2, w3, b3)


def _edge_mlp(bv, gs, gd, w1c, b1, w2, b2, w3, b3):
    g = EP // BE
    return pl.pallas_call(
        _edge_body, grid=(g,),
        in_specs=[_rows(BE, 192), _rows(BE, 128), _rows(BE, 128),
                  _full((192, 128)), _full((1, 128)), _full((128, 128)),
                  _full((1, 128)), _full((128, 192)), _full((1, 192))],
        out_specs=[_rows(BE, 128), _rows(BE, 192)],
        out_shape=[jax.ShapeDtypeStruct((EP, 128), F32),
                   jax.ShapeDtypeStruct((EP, 192), F32)],
    )(bv, gs, gd, w1c, b1, w2, b2, w3, b3)


def _node_update(av, sv, acc0, acc1, d0, d1, ws):
    g = NP // BN
    return pl.pallas_call(
        _node_upd_body, grid=(g,),
        in_specs=[_rows(BN, 192), _rows(BN, 192), _rows(BN, 128), _rows(BN, 128),
                  _rows(BN, 128), _rows(BN, 128),
                  _full((192, 128)), _full((192, 128)), _full((192, 128)),
                  _full((1, 128)), _full((128, 128)), _full((1, 128)),
                  _full((128, 192)), _full((1, 192)),
                  _full((192, 128)), _full((192, 128)), _full((192, 128)),
                  _full((1, 128)), _full((128, 128)), _full((1, 128)),
                  _full((128, 192)), _full((1, 192)),
                  _full((128, 192)), _full((1, 192)),
                  _full((192, 128)), _full((192, 128)), _full((192, 128))],
        out_specs=[_rows(BN, 192), _rows(BN, 192), _rows(BN, 128), _rows(BN, 128)],
        out_shape=[jax.ShapeDtypeStruct((NP, 192), F32),
                   jax.ShapeDtypeStruct((NP, 192), F32),
                   jax.ShapeDtypeStruct((NP, 128), F32),
                   jax.ShapeDtypeStruct((NP, 128), F32)],
    )(av, sv, acc0, acc1, d0, d1, *ws)


def _node_update_last(av, sv, acc0, acc1, d0, d1, ws):
    g = NP // BN
    return pl.pallas_call(
        _node_upd_last_body, grid=(g,),
        in_specs=[_rows(BN, 192), _rows(BN, 192), _rows(BN, 128), _rows(BN, 128),
                  _rows(BN, 128), _rows(BN, 128),
                  _full((192, 128)), _full((192, 128)), _full((192, 128)),
                  _full((1, 128)), _full((128, 128)), _full((1, 128)),
                  _full((128, 192)), _full((1, 192)),
                  _full((192, 128)), _full((192, 128)), _full((192, 128)),
                  _full((1, 128)), _full((128, 128)), _full((1, 128)),
                  _full((128, 192)), _full((1, 192)),
                  _full((128, 192)), _full((1, 192))],
        out_specs=[_rows(BN, 192), _rows(BN, 192)],
        out_shape=[jax.ShapeDtypeStruct((NP, 192), F32),
                   jax.ShapeDtypeStruct((NP, 192), F32)],
    )(av, sv, acc0, acc1, d0, d1, *ws[:-3])


def _settf_a(x, oh, wp, bp, wl, bl, aV, rows, bn):
    g = rows // bn
    return pl.pallas_call(
        _settf_a_body, grid=(g,),
        in_specs=[_rows(bn, 192), _rows(bn, 32), _full((192, 128)), _full((1, 128)),
                  _full((128, 128)), _full((1, 128)), _full((128, 1))],
        out_specs=[_rows(bn, 128), _rows(bn, 1), _full((1, 32))],
        out_shape=[jax.ShapeDtypeStruct((rows, 128), F32),
                   jax.ShapeDtypeStruct((rows, 1), F32),
                   jax.ShapeDtypeStruct((1, 32), F32)],
    )(x, oh, wp, bp, wl, bl, aV)


def _settf_b(h, s, oh, smax, rows, bn):
    g = rows // bn
    return pl.pallas_call(
        _settf_b_body, grid=(g,),
        in_specs=[_rows(bn, 128), _rows(bn, 1), _rows(bn, 32), _full((1, 32))],
        out_specs=[_full((32, 128)), _full((32, 1)), _full((32, 128)), _full((32, 1))],
        out_shape=[jax.ShapeDtypeStruct((32, 128), F32),
                   jax.ShapeDtypeStruct((32, 1), F32),
                   jax.ShapeDtypeStruct((32, 128), F32),
                   jax.ShapeDtypeStruct((32, 1), F32)],
    )(h, s, oh, smax)


def _finalize(cche, a4, b4, clW, clB, chWab, chB):
    return pl.pallas_call(
        _finalize_body, grid=(1,),
        in_specs=[_full((32, 64)),
                  _full((32, 128)), _full((32, 1)), _full((32, 128)), _full((32, 1)),
                  _full((32, 128)), _full((32, 1)), _full((32, 128)), _full((32, 1)),
                  _full((64, 1)), _full((1, 1)), _full((512, 256)), _full((1, 256))],
        out_specs=_full((32, 256)),
        out_shape=jax.ShapeDtypeStruct((32, 256), F32),
    )(cche, *a4, *b4, clW, clB, chWab, chB)


def _final_out(oh, sv, t, chWs):
    g = NP // BN
    return pl.pallas_call(
        _final_out_body, grid=(g,),
        in_specs=[_rows(BN, 32), _rows(BN, 192), _full((32, 256)), _full((192, 256))],
        out_specs=_rows(BN, 256),
        out_shape=jax.ShapeDtypeStruct((NP, 256), F32),
    )(oh, sv, t, chWs)


def kernel(atom_vec_embedded, bond_vec_embedded, state_vec_embedded, atom_fea,
           comp_w, params, edge_index, batch):
    p = params

    def padr(x, n):
        return jnp.pad(x, ((0, n - x.shape[0]), (0, 0)))

    a80 = padr(atom_vec_embedded, NP)
    s64 = padr(state_vec_embedded, NP)
    afea = padr(atom_fea, NP)
    cw = padr(comp_w.reshape(-1, 1), NP)
    b100 = padr(bond_vec_embedded, EP)
    batch_p = jnp.pad(batch, (0, NP - N), constant_values=B)
    oh = (batch_p[:, None] == jnp.arange(B, dtype=batch.dtype)).astype(F32)
    src_p = jnp.pad(edge_index[0], (0, EP - E), constant_values=N)
    dst_p = jnp.pad(edge_index[1], (0, EP - E), constant_values=N)
    zeros128 = jnp.zeros((NP, 128), F32)
    ones128 = jnp.ones((SCH, 128), F32)

    r2 = lambda b: b.reshape(1, -1)
    e1a, e1b, e1c, e1d = (p['eW1'][0:192], p['eW1'][192:384], p['eW1'][384:576],
                          p['eW1'][576:768])
    vW1a, vW1b, vW1c = p['vW1'][0:192], p['vW1'][192:384], p['vW1'][384:576]
    uW1a, uW1b, uW1c = p['uW1'][0:192], p['uW1'][192:384], p['uW1'][384:576]

    av, sv, P, Q, cche = _node_init(
        a80, s64, afea, cw, oh,
        (p['aW1'], r2(p['aB1']), p['aW2'], r2(p['aB2']),
         p['sW1'], r2(p['sB1']), p['sW2'], r2(p['sB2']),
         p['cW'], r2(p['cB']), e1a, e1b, e1d))

    oh128 = jnp.pad(oh, ((0, 0), (0, 128 - B)))
    dst2ds = dst_p.reshape(EP // SCH, SCH)
    deg2, ohsrc_raw = _sc_prep(dst2ds, src_p, oh128, zeros128, ones128)
    ohsrc = ohsrc_raw[:, 0:B]
    d0, d1 = deg2[:NP], deg2[NP:]

    upd_ws = (vW1a, vW1b, vW1c, r2(p['vB1']), p['vW2'], r2(p['vB2']),
              p['vW3'], r2(p['vB3']),
              uW1a, uW1b, uW1c, r2(p['uB1']), p['uW2'], r2(p['uB2']),
              p['uW3'], r2(p['uB3']), p['eW3'], r2(p['eB3']), e1a, e1b, e1d)

    ew = (e1c, r2(p['eB1']), p['eW2'], r2(p['eB2']), p['eW3'], r2(p['eB3']))
    bw = (p['bW1'], r2(p['bB1']), p['bW2'], r2(p['bB2']))
    bv = None
    for it in range(3):
        gs, gd = _sc_gather2(P, Q, src_p, dst_p)
        if it == 0:
            h2, bv = _edge1_mlp(b100, bw, gs, gd, *ew)
        else:
            h2, bv = _edge_mlp(bv, gs, gd, *ew)
        acc2 = _sc_scatter(h2, dst2ds, zeros128)
        if it < 2:
            av, sv, P, Q = _node_update(av, sv, acc2[:NP], acc2[NP:], d0, d1, upd_ws)
        else:
            av, sv = _node_update_last(av, sv, acc2[:NP], acc2[NP:], d0, d1, upd_ws)

    h_a, s_a, smax_a = _settf_a(av, oh, p['asWp'], r2(p['asBp']), p['asWl'],
                                r2(p['asBl']), p['asA'], NP, BN)
    a4 = _settf_b(h_a, s_a, oh, smax_a, NP, BN)
    h_b, s_b, smax_b = _settf_a(bv, ohsrc, p['bsWp'], r2(p['bsBp']), p['bsWl'],
                                r2(p['bsBl']), p['bsA'], EP, BE)
    b4 = _settf_b(h_b, s_b, ohsrc, smax_b, EP, BE)

    t = _finalize(cche, a4, b4, p['clW'], r2(p['clB']), p['chW'][0:512],
                  r2(p['chB']))
    out = _final_out(oh, sv, t, p['chW'][512:704])
    return out[:N]
